# baseline (speedup 1.0000x reference)
import functools

import jax
import jax.numpy as jnp
from jax import lax
from jax.experimental import pallas as pl
from jax.experimental.pallas import tpu as pltpu
from jax.experimental.pallas import tpu_sc as plsc

_NC = 2
_NS = 16
_NW = _NC * _NS


def _edge_emb_one(attr_pk, Wbig, be_l, E):
    M, KP = attr_pk.shape
    DP = Wbig.shape[1]
    D = DP // 16
    BM = 200
    nblk = M // BM

    def body(a_ref, w_ref, b_ref, o_ref):
        o_ref[...] = (
            jnp.dot(a_ref[...], w_ref[...], preferred_element_type=jnp.float32)
            + b_ref[0]
        )

    out = pl.pallas_call(
        body,
        grid=(nblk,),
        in_specs=[
            pl.BlockSpec((BM, KP), lambda i: (i, 0)),
            pl.BlockSpec((KP, DP), lambda i: (0, 0)),
            pl.BlockSpec((1, DP), lambda i: (0, 0)),
        ],
        out_specs=pl.BlockSpec((BM, DP), lambda i: (i, 0)),
        out_shape=jax.ShapeDtypeStruct((M, DP), jnp.float32),
    )(attr_pk, Wbig, jnp.tile(be_l, 16).reshape(1, DP))
    return out.reshape(E, D)


def _mlp_layer(h, agg2, eps2, W1, b1, g1, bt1, m1, v1,
               W2, b2, gbn, bbn, mbn, vbn, l, last):
    N, D = h.shape
    H = W1.shape[2]
    BN = 2000
    nblk = N // BN
    eps2_shape = eps2.shape

    def body(h_ref, a_ref, e_ref, w1_ref, b1_ref, g1_ref, bt1_ref, m1_ref,
             v1_ref, w2_ref, b2_ref, gbn_ref, bbn_ref, mbn_ref, vbn_ref,
             o_ref):
        s1 = g1_ref[l] * lax.rsqrt(v1_ref[l] + 1e-5)
        c1 = (b1_ref[l] - m1_ref[l]) * s1 + bt1_ref[l]
        s2 = gbn_ref[l] * lax.rsqrt(vbn_ref[l] + 1e-5)
        c2 = (b2_ref[l] - mbn_ref[l]) * s2 + bbn_ref[l]
        t = h_ref[...] * (1.0 + e_ref[l, 0]) + a_ref[0] + a_ref[1]
        t = jnp.dot(t, w1_ref[0], preferred_element_type=jnp.float32)
        t = jnp.maximum(t * s1 + c1, 0.0)
        t = jnp.dot(t, w2_ref[0], preferred_element_type=jnp.float32)
        t = t * s2 + c2
        if not last:
            t = jnp.maximum(t, 0.0)
        o_ref[...] = t

    L = W1.shape[0]
    full = lambda i: (0, 0)
    return pl.pallas_call(
        body,
        grid=(nblk,),
        in_specs=[
            pl.BlockSpec((BN, D), lambda i: (i, 0)),
            pl.BlockSpec((2, BN, D), lambda i: (0, i, 0)),
            pl.BlockSpec(eps2_shape, full),
            pl.BlockSpec((1, D, H), lambda i: (l, 0, 0)),
            pl.BlockSpec((L, H), full),
            pl.BlockSpec((L, H), full),
            pl.BlockSpec((L, H), full),
            pl.BlockSpec((L, H), full),
            pl.BlockSpec((L, H), full),
            pl.BlockSpec((1, H, D), lambda i: (l, 0, 0)),
            pl.BlockSpec((L, D), full),
            pl.BlockSpec((L, D), full),
            pl.BlockSpec((L, D), full),
            pl.BlockSpec((L, D), full),
            pl.BlockSpec((L, D), full),
        ],
        out_specs=pl.BlockSpec((BN, D), lambda i: (i, 0)),
        out_shape=jax.ShapeDtypeStruct((N, D), jnp.float32),
    )(h, agg2, eps2, W1, b1, g1, bt1, m1, v1, W2, b2, gbn, bbn, mbn, vbn)


def _sc_embed(node_emb, idx_pad, B):
    V, D = node_emb.shape
    per_w = B // _NW
    CH = 80
    nch = per_w // CH
    rem = per_w - nch * CH
    mesh = plsc.VectorSubcoreMesh(core_axis_name="c", subcore_axis_name="s")

    @functools.partial(
        pl.kernel,
        out_type=jax.ShapeDtypeStruct((B, D), jnp.float32),
        mesh=mesh,
        scratch_types=[
            pltpu.VMEM((per_w,), jnp.int32),
            pltpu.VMEM((CH, D), jnp.float32),
            pltpu.SemaphoreType.DMA,
        ],
    )
    def k(table_hbm, idx_hbm, out_hbm, idx_v, rows_v, sem):
        cid = lax.axis_index("c")
        sid = lax.axis_index("s")
        wid = sid * _NC + cid
        base = wid * per_w
        pltpu.sync_copy(idx_hbm.at[pl.ds(base, per_w)], idx_v)
        for j in range(nch):
            pltpu.async_copy(
                table_hbm.at[idx_v.at[pl.ds(j * CH, CH)]], rows_v, sem
            ).wait()
            pltpu.sync_copy(rows_v, out_hbm.at[pl.ds(base + j * CH, CH)])
        if rem:
            pltpu.async_copy(
                table_hbm.at[idx_v.at[pl.ds(nch * CH, rem)]],
                rows_v.at[pl.ds(0, rem)], sem
            ).wait()
            pltpu.sync_copy(rows_v.at[pl.ds(0, rem)],
                            out_hbm.at[pl.ds(base + nch * CH, rem)])

    return k(node_emb, idx_pad)


def _sc_aggregate(h, emb, row, col):
    N, D = h.shape
    E, _ = emb.shape
    EPW = E // _NW
    HPW = EPW // 2
    C = 40
    nch = HPW // C
    assert nch * C == HPW and nch % 2 == 1 and nch >= 5
    gend = (nch - 5) // 2 + 1
    rps = (N // _NS) & ~7
    rem_n = N - rps * _NS
    ND2 = D // 16
    mesh = plsc.VectorSubcoreMesh(core_axis_name="c", subcore_axis_name="s")

    @functools.partial(
        pl.kernel,
        out_type=jax.ShapeDtypeStruct((_NC, N, D), jnp.float32),
        mesh=mesh,
        scratch_types=[
            pltpu.VMEM((HPW,), jnp.int32),
            pltpu.VMEM((HPW,), jnp.int32),
            pltpu.VMEM((C, D), jnp.float32),
            pltpu.VMEM((C, D), jnp.float32),
            pltpu.VMEM((C, D), jnp.float32),
            pltpu.VMEM((C, D), jnp.float32),
            pltpu.VMEM((C, D), jnp.float32),
            pltpu.VMEM((C, D), jnp.float32),
            pltpu.VMEM_SHARED((N, D), jnp.float32),
            pltpu.SemaphoreType.DMA,
            pltpu.SemaphoreType.DMA,
            pltpu.SemaphoreType.DMA,
            pltpu.SemaphoreType.DMA,
        ],
    )
    def k(h_hbm, emb_hbm, row_hbm, col_hbm, out_hbm,
          row_v, col_v, h0, h1, e0, e1, o0, o1, agg_sh,
          semEH0, semEH1, semS0, semS1):
        cid = lax.axis_index("c")
        sid = lax.axis_index("s")
        wid = sid * _NC + cid
        ebase = wid * EPW
        hs = (h0, h1)
        es = (e0, e1)
        os_ = (o0, o1)
        semEH = (semEH0, semEH1)
        semS = (semS0, semS1)

        def load_idx(hoff):
            pltpu.sync_copy(row_hbm.at[pl.ds(ebase + hoff, HPW)], row_v)
            pltpu.sync_copy(col_hbm.at[pl.ds(ebase + hoff, HPW)], col_v)

        def issue(hoff, i, s):
            pltpu.async_copy(emb_hbm.at[pl.ds(ebase + hoff + i * C, C), :],
                             es[s], semEH[s])
            pltpu.async_copy(h_hbm.at[row_v.at[pl.ds(i * C, C)]],
                             hs[s], semEH[s])

        def wait_eh(s):
            pltpu.make_async_copy(emb_hbm.at[pl.ds(ebase, C), :],
                                  es[s], semEH[s]).wait()
            pltpu.make_async_copy(h_hbm.at[pl.ds(0, C)], hs[s],
                                  semEH[s]).wait()

        def wait_s(s):
            pltpu.make_async_copy(h_hbm.at[pl.ds(0, C)], os_[s],
                                  semS[s]).wait()

        def compute(s):
            def ebody(e, _):
                for d in range(ND2):
                    sl = pl.ds(d * 16, 16)
                    os_[s][e, sl] = jnp.maximum(es[s][e, sl] + hs[s][e, sl],
                                                0.0)
                return 0
            lax.fori_loop(0, C, ebody, 0)

        def scatter(i, s):
            pltpu.async_copy(os_[s], agg_sh.at[col_v.at[pl.ds(i * C, C)]],
                             semS[s], add=True)

        def pipeline(hoff):
            for s in range(2):
                wait_eh(s)
                compute(s)
                issue(hoff, 2 + s, s)
                scatter(s, s)

            def pbody(g, _):
                i = 2 * g
                for s in range(2):
                    wait_eh(s)
                    wait_s(s)
                    compute(s)
                    issue(hoff, i + 2 + s, s)
                    scatter(i + s, s)
                return 0
            lax.fori_loop(1, gend, pbody, 0)

            wait_eh(0)
            wait_s(0)
            compute(0)
            issue(hoff, nch - 1, 0)
            scatter(nch - 3, 0)
            wait_eh(1)
            wait_s(1)
            compute(1)
            scatter(nch - 2, 1)
            wait_eh(0)
            wait_s(0)
            compute(0)
            scatter(nch - 1, 0)
            for s in range(2):
                wait_s(s)

        load_idx(0)
        issue(0, 0, 0)
        issue(0, 1, 1)

        def zbody(e, _):
            for d in range(ND2):
                o0[e, pl.ds(d * 16, 16)] = jnp.zeros((16,), jnp.float32)
            return 0
        lax.fori_loop(0, C, zbody, 0)
        nzc = rps // C
        for j in range(nzc):
            pltpu.sync_copy(o0, agg_sh.at[pl.ds(sid * rps + j * C, C), :])
        zrem = rps - nzc * C
        if zrem:
            pltpu.sync_copy(o0.at[pl.ds(0, zrem), :],
                            agg_sh.at[pl.ds(sid * rps + nzc * C, zrem), :])
        if rem_n:
            @pl.when(sid == _NS - 1)
            def _():
                pltpu.sync_copy(o0.at[pl.ds(0, rem_n), :],
                                agg_sh.at[pl.ds(_NS * rps, rem_n), :])
        plsc.subcore_barrier()

        pipeline(0)

        load_idx(HPW)
        issue(HPW, 0, 0)
        issue(HPW, 1, 1)
        pipeline(HPW)
        plsc.subcore_barrier()

        for j in range(nzc):
            pltpu.sync_copy(agg_sh.at[pl.ds(sid * rps + j * C, C), :],
                            out_hbm.at[cid, pl.ds(sid * rps + j * C, C), :])
        if zrem:
            pltpu.sync_copy(
                agg_sh.at[pl.ds(sid * rps + nzc * C, zrem), :],
                out_hbm.at[cid, pl.ds(sid * rps + nzc * C, zrem), :])
        if rem_n:
            @pl.when(sid == _NS - 1)
            def _():
                pltpu.sync_copy(
                    agg_sh.at[pl.ds(_NS * rps, rem_n), :],
                    out_hbm.at[cid, pl.ds(_NS * rps, rem_n), :])

    return k(h, emb, row, col)


def kernel(x, edge_index, edge_attr, batch, node_emb, We, be, eps,
           W1, b1, g1, bt1, m1, v1, W2, b2, gbn, bbn, mbn, vbn):
    N = x.shape[0]
    L, K, D = We.shape
    E = edge_index.shape[1]

    row = edge_index[0].astype(jnp.int32)
    col = edge_index[1].astype(jnp.int32)

    B = ((N + 8 * _NW - 1) // (8 * _NW)) * (8 * _NW)
    xi = jnp.pad(x[:, 0].astype(jnp.int32), (0, B - N))
    h = _sc_embed(node_emb.astype(jnp.float32), xi, B)[:N]

    attr_pk = edge_attr.reshape(E // 16, 16 * K)
    ar = jnp.arange(16 * K)
    jj, kk = ar // K, ar % K
    embs = []
    for l in range(L):
        Wbig = (jnp.zeros((16 * K, 16, D), jnp.float32)
                .at[ar, jj].set(We[l][kk]).reshape(16 * K, 16 * D))
        embs.append(_edge_emb_one(attr_pk, Wbig, be[l], E))

    eps2 = eps.reshape(L, 1)
    for l in range(L):
        agg2 = _sc_aggregate(h, embs[l], row, col)
        h = _mlp_layer(h, agg2, eps2, W1, b1, g1, bt1, m1, v1,
                       W2, b2, gbn, bbn, mbn, vbn, l, last=(l == L - 1))
    return h


# speedup vs baseline: 1.2524x; 1.2524x over previous
import functools

import jax
import jax.numpy as jnp
from jax import lax
from jax.experimental import pallas as pl
from jax.experimental.pallas import tpu as pltpu
from jax.experimental.pallas import tpu_sc as plsc

_NC = 2
_NS = 16
_NW = _NC * _NS


def _edge_emb_one(edge_attr, We_l, be_l):
    E, K = edge_attr.shape
    D = We_l.shape[1]
    BE = 8000
    nblk = E // BE

    def body(a_ref, w_ref, b_ref, o_ref):
        o_ref[...] = (
            jnp.dot(a_ref[...], w_ref[...], preferred_element_type=jnp.float32)
            + b_ref[0]
        )

    return pl.pallas_call(
        body,
        grid=(nblk,),
        in_specs=[
            pl.BlockSpec((BE, K), lambda i: (i, 0)),
            pl.BlockSpec((K, D), lambda i: (0, 0)),
            pl.BlockSpec((1, D), lambda i: (0, 0)),
        ],
        out_specs=pl.BlockSpec((BE, D), lambda i: (i, 0)),
        out_shape=jax.ShapeDtypeStruct((E, D), jnp.float32),
    )(edge_attr, We_l, be_l.reshape(1, D))


def _mlp_layer(h, agg2, eps2, W1, b1, g1, bt1, m1, v1,
               W2, b2, gbn, bbn, mbn, vbn, l, last):
    N, D = h.shape
    H = W1.shape[2]
    BN = 2000
    nblk = N // BN
    eps2_shape = eps2.shape

    def body(h_ref, a_ref, e_ref, w1_ref, b1_ref, g1_ref, bt1_ref, m1_ref,
             v1_ref, w2_ref, b2_ref, gbn_ref, bbn_ref, mbn_ref, vbn_ref,
             o_ref):
        s1 = g1_ref[l] * lax.rsqrt(v1_ref[l] + 1e-5)
        c1 = (b1_ref[l] - m1_ref[l]) * s1 + bt1_ref[l]
        s2 = gbn_ref[l] * lax.rsqrt(vbn_ref[l] + 1e-5)
        c2 = (b2_ref[l] - mbn_ref[l]) * s2 + bbn_ref[l]
        t = h_ref[...] * (1.0 + e_ref[l, 0]) + a_ref[0] + a_ref[1]
        t = jnp.dot(t, w1_ref[0], preferred_element_type=jnp.float32)
        t = jnp.maximum(t * s1 + c1, 0.0)
        t = jnp.dot(t, w2_ref[0], preferred_element_type=jnp.float32)
        t = t * s2 + c2
        if not last:
            t = jnp.maximum(t, 0.0)
        o_ref[...] = t

    L = W1.shape[0]
    full = lambda i: (0, 0)
    return pl.pallas_call(
        body,
        grid=(nblk,),
        in_specs=[
            pl.BlockSpec((BN, D), lambda i: (i, 0)),
            pl.BlockSpec((2, BN, D), lambda i: (0, i, 0)),
            pl.BlockSpec(eps2_shape, full),
            pl.BlockSpec((1, D, H), lambda i: (l, 0, 0)),
            pl.BlockSpec((L, H), full),
            pl.BlockSpec((L, H), full),
            pl.BlockSpec((L, H), full),
            pl.BlockSpec((L, H), full),
            pl.BlockSpec((L, H), full),
            pl.BlockSpec((1, H, D), lambda i: (l, 0, 0)),
            pl.BlockSpec((L, D), full),
            pl.BlockSpec((L, D), full),
            pl.BlockSpec((L, D), full),
            pl.BlockSpec((L, D), full),
            pl.BlockSpec((L, D), full),
        ],
        out_specs=pl.BlockSpec((BN, D), lambda i: (i, 0)),
        out_shape=jax.ShapeDtypeStruct((N, D), jnp.float32),
    )(h, agg2, eps2, W1, b1, g1, bt1, m1, v1, W2, b2, gbn, bbn, mbn, vbn)


def _sc_embed(node_emb, idx_pad, B):
    V, D = node_emb.shape
    per_w = B // _NW
    CH = 80
    nch = per_w // CH
    rem = per_w - nch * CH
    mesh = plsc.VectorSubcoreMesh(core_axis_name="c", subcore_axis_name="s")

    @functools.partial(
        pl.kernel,
        out_type=jax.ShapeDtypeStruct((B, D), jnp.float32),
        mesh=mesh,
        scratch_types=[
            pltpu.VMEM((per_w,), jnp.int32),
            pltpu.VMEM((CH, D), jnp.float32),
            pltpu.SemaphoreType.DMA,
        ],
    )
    def k(table_hbm, idx_hbm, out_hbm, idx_v, rows_v, sem):
        cid = lax.axis_index("c")
        sid = lax.axis_index("s")
        wid = sid * _NC + cid
        base = wid * per_w
        pltpu.sync_copy(idx_hbm.at[pl.ds(base, per_w)], idx_v)
        for j in range(nch):
            pltpu.async_copy(
                table_hbm.at[idx_v.at[pl.ds(j * CH, CH)]], rows_v, sem
            ).wait()
            pltpu.sync_copy(rows_v, out_hbm.at[pl.ds(base + j * CH, CH)])
        if rem:
            pltpu.async_copy(
                table_hbm.at[idx_v.at[pl.ds(nch * CH, rem)]],
                rows_v.at[pl.ds(0, rem)], sem
            ).wait()
            pltpu.sync_copy(rows_v.at[pl.ds(0, rem)],
                            out_hbm.at[pl.ds(base + nch * CH, rem)])

    return k(node_emb, idx_pad)


def _sc_aggregate(h, emb, row, col):
    N, D = h.shape
    E, _ = emb.shape
    EPW = E // _NW
    HPW = EPW // 2
    C = 40
    nch = HPW // C
    assert nch * C == HPW and nch % 2 == 1 and nch >= 5
    gend = (nch - 5) // 2 + 1
    rps = (N // _NS) & ~7
    rem_n = N - rps * _NS
    ND2 = D // 16
    mesh = plsc.VectorSubcoreMesh(core_axis_name="c", subcore_axis_name="s")

    @functools.partial(
        pl.kernel,
        out_type=jax.ShapeDtypeStruct((_NC, N, D), jnp.float32),
        mesh=mesh,
        scratch_types=[
            pltpu.VMEM((HPW,), jnp.int32),
            pltpu.VMEM((HPW,), jnp.int32),
            pltpu.VMEM((C, D), jnp.float32),
            pltpu.VMEM((C, D), jnp.float32),
            pltpu.VMEM((C, D), jnp.float32),
            pltpu.VMEM((C, D), jnp.float32),
            pltpu.VMEM((C, D), jnp.float32),
            pltpu.VMEM((C, D), jnp.float32),
            pltpu.VMEM_SHARED((N, D), jnp.float32),
            pltpu.SemaphoreType.DMA,
            pltpu.SemaphoreType.DMA,
            pltpu.SemaphoreType.DMA,
            pltpu.SemaphoreType.DMA,
        ],
    )
    def k(h_hbm, emb_hbm, row_hbm, col_hbm, out_hbm,
          row_v, col_v, h0, h1, e0, e1, o0, o1, agg_sh,
          semEH0, semEH1, semS0, semS1):
        cid = lax.axis_index("c")
        sid = lax.axis_index("s")
        wid = sid * _NC + cid
        ebase = wid * EPW
        hs = (h0, h1)
        es = (e0, e1)
        os_ = (o0, o1)
        semEH = (semEH0, semEH1)
        semS = (semS0, semS1)

        def load_idx(hoff):
            pltpu.sync_copy(row_hbm.at[pl.ds(ebase + hoff, HPW)], row_v)
            pltpu.sync_copy(col_hbm.at[pl.ds(ebase + hoff, HPW)], col_v)

        def issue(hoff, i, s):
            pltpu.async_copy(emb_hbm.at[pl.ds(ebase + hoff + i * C, C), :],
                             es[s], semEH[s])
            pltpu.async_copy(h_hbm.at[row_v.at[pl.ds(i * C, C)]],
                             hs[s], semEH[s])

        def wait_eh(s):
            pltpu.make_async_copy(emb_hbm.at[pl.ds(ebase, C), :],
                                  es[s], semEH[s]).wait()
            pltpu.make_async_copy(h_hbm.at[pl.ds(0, C)], hs[s],
                                  semEH[s]).wait()

        def wait_s(s):
            pltpu.make_async_copy(h_hbm.at[pl.ds(0, C)], os_[s],
                                  semS[s]).wait()

        def compute(s):
            def ebody(e, _):
                for d in range(ND2):
                    sl = pl.ds(d * 16, 16)
                    os_[s][e, sl] = jnp.maximum(es[s][e, sl] + hs[s][e, sl],
                                                0.0)
                return 0
            lax.fori_loop(0, C, ebody, 0)

        def scatter(i, s):
            pltpu.async_copy(os_[s], agg_sh.at[col_v.at[pl.ds(i * C, C)]],
                             semS[s], add=True)

        def pipeline(hoff):
            for s in range(2):
                wait_eh(s)
                compute(s)
                issue(hoff, 2 + s, s)
                scatter(s, s)

            def pbody(g, _):
                i = 2 * g
                for s in range(2):
                    wait_eh(s)
                    wait_s(s)
                    compute(s)
                    issue(hoff, i + 2 + s, s)
                    scatter(i + s, s)
                return 0
            lax.fori_loop(1, gend, pbody, 0)

            wait_eh(0)
            wait_s(0)
            compute(0)
            issue(hoff, nch - 1, 0)
            scatter(nch - 3, 0)
            wait_eh(1)
            wait_s(1)
            compute(1)
            scatter(nch - 2, 1)
            wait_eh(0)
            wait_s(0)
            compute(0)
            scatter(nch - 1, 0)
            for s in range(2):
                wait_s(s)

        load_idx(0)
        issue(0, 0, 0)
        issue(0, 1, 1)

        def zbody(e, _):
            for d in range(ND2):
                o0[e, pl.ds(d * 16, 16)] = jnp.zeros((16,), jnp.float32)
            return 0
        lax.fori_loop(0, C, zbody, 0)
        nzc = rps // C
        for j in range(nzc):
            pltpu.sync_copy(o0, agg_sh.at[pl.ds(sid * rps + j * C, C), :])
        zrem = rps - nzc * C
        if zrem:
            pltpu.sync_copy(o0.at[pl.ds(0, zrem), :],
                            agg_sh.at[pl.ds(sid * rps + nzc * C, zrem), :])
        if rem_n:
            @pl.when(sid == _NS - 1)
            def _():
                pltpu.sync_copy(o0.at[pl.ds(0, rem_n), :],
                                agg_sh.at[pl.ds(_NS * rps, rem_n), :])
        plsc.subcore_barrier()

        pipeline(0)

        load_idx(HPW)
        issue(HPW, 0, 0)
        issue(HPW, 1, 1)
        pipeline(HPW)
        plsc.subcore_barrier()

        for j in range(nzc):
            pltpu.sync_copy(agg_sh.at[pl.ds(sid * rps + j * C, C), :],
                            out_hbm.at[cid, pl.ds(sid * rps + j * C, C), :])
        if zrem:
            pltpu.sync_copy(
                agg_sh.at[pl.ds(sid * rps + nzc * C, zrem), :],
                out_hbm.at[cid, pl.ds(sid * rps + nzc * C, zrem), :])
        if rem_n:
            @pl.when(sid == _NS - 1)
            def _():
                pltpu.sync_copy(
                    agg_sh.at[pl.ds(_NS * rps, rem_n), :],
                    out_hbm.at[cid, pl.ds(_NS * rps, rem_n), :])

    return k(h, emb, row, col)


def kernel(x, edge_index, edge_attr, batch, node_emb, We, be, eps,
           W1, b1, g1, bt1, m1, v1, W2, b2, gbn, bbn, mbn, vbn):
    N = x.shape[0]
    L, K, D = We.shape
    E = edge_index.shape[1]

    row = edge_index[0].astype(jnp.int32)
    col = edge_index[1].astype(jnp.int32)

    B = ((N + 8 * _NW - 1) // (8 * _NW)) * (8 * _NW)
    xi = jnp.pad(x[:, 0].astype(jnp.int32), (0, B - N))
    h = _sc_embed(node_emb.astype(jnp.float32), xi, B)[:N]

    embs = [_edge_emb_one(edge_attr, We[l], be[l]) for l in range(L)]

    eps2 = eps.reshape(L, 1)
    for l in range(L):
        agg2 = _sc_aggregate(h, embs[l], row, col)
        h = _mlp_layer(h, agg2, eps2, W1, b1, g1, bt1, m1, v1,
                       W2, b2, gbn, bbn, mbn, vbn, l, last=(l == L - 1))
    return h


# speedup vs baseline: 1.3571x; 1.0836x over previous
import functools

import jax
import jax.numpy as jnp
from jax import lax
from jax.experimental import pallas as pl
from jax.experimental.pallas import tpu as pltpu
from jax.experimental.pallas import tpu_sc as plsc

_NC = 2
_NS = 16
_NW = _NC * _NS


def _edge_emb_one(edge_attr, We_l, be_l):
    E, K = edge_attr.shape
    D = We_l.shape[1]
    BE = 8000
    nblk = E // BE

    def body(a_ref, w_ref, b_ref, o_ref):
        a = a_ref[...].astype(jnp.float32)
        o_ref[...] = (
            jnp.dot(a, w_ref[...], preferred_element_type=jnp.float32)
            + b_ref[0]
        )

    return pl.pallas_call(
        body,
        grid=(nblk,),
        in_specs=[
            pl.BlockSpec((BE, K), lambda i: (i, 0)),
            pl.BlockSpec((K, D), lambda i: (0, 0)),
            pl.BlockSpec((1, D), lambda i: (0, 0)),
        ],
        out_specs=pl.BlockSpec((BE, D), lambda i: (i, 0)),
        out_shape=jax.ShapeDtypeStruct((E, D), jnp.float32),
    )(edge_attr, We_l, be_l.reshape(1, D))


def _mlp_layer(h, agg2, eps2, W1, b1, g1, bt1, m1, v1,
               W2, b2, gbn, bbn, mbn, vbn, l, last):
    N, D = h.shape
    H = W1.shape[2]
    BN = 2000
    nblk = N // BN
    eps2_shape = eps2.shape

    def body(h_ref, a_ref, e_ref, w1_ref, b1_ref, g1_ref, bt1_ref, m1_ref,
             v1_ref, w2_ref, b2_ref, gbn_ref, bbn_ref, mbn_ref, vbn_ref,
             o_ref):
        s1 = g1_ref[l] * lax.rsqrt(v1_ref[l] + 1e-5)
        c1 = (b1_ref[l] - m1_ref[l]) * s1 + bt1_ref[l]
        s2 = gbn_ref[l] * lax.rsqrt(vbn_ref[l] + 1e-5)
        c2 = (b2_ref[l] - mbn_ref[l]) * s2 + bbn_ref[l]
        t = h_ref[...] * (1.0 + e_ref[l, 0]) + a_ref[0] + a_ref[1]
        t = jnp.dot(t, w1_ref[0], preferred_element_type=jnp.float32)
        t = jnp.maximum(t * s1 + c1, 0.0)
        t = jnp.dot(t, w2_ref[0], preferred_element_type=jnp.float32)
        t = t * s2 + c2
        if not last:
            t = jnp.maximum(t, 0.0)
        o_ref[...] = t

    L = W1.shape[0]
    full = lambda i: (0, 0)
    return pl.pallas_call(
        body,
        grid=(nblk,),
        in_specs=[
            pl.BlockSpec((BN, D), lambda i: (i, 0)),
            pl.BlockSpec((2, BN, D), lambda i: (0, i, 0)),
            pl.BlockSpec(eps2_shape, full),
            pl.BlockSpec((1, D, H), lambda i: (l, 0, 0)),
            pl.BlockSpec((L, H), full),
            pl.BlockSpec((L, H), full),
            pl.BlockSpec((L, H), full),
            pl.BlockSpec((L, H), full),
            pl.BlockSpec((L, H), full),
            pl.BlockSpec((1, H, D), lambda i: (l, 0, 0)),
            pl.BlockSpec((L, D), full),
            pl.BlockSpec((L, D), full),
            pl.BlockSpec((L, D), full),
            pl.BlockSpec((L, D), full),
            pl.BlockSpec((L, D), full),
        ],
        out_specs=pl.BlockSpec((BN, D), lambda i: (i, 0)),
        out_shape=jax.ShapeDtypeStruct((N, D), jnp.float32),
    )(h, agg2, eps2, W1, b1, g1, bt1, m1, v1, W2, b2, gbn, bbn, mbn, vbn)


def _sc_embed(node_emb, idx_pad, B):
    V, D = node_emb.shape
    per_w = B // _NW
    CH = 80
    nch = per_w // CH
    rem = per_w - nch * CH
    mesh = plsc.VectorSubcoreMesh(core_axis_name="c", subcore_axis_name="s")

    @functools.partial(
        pl.kernel,
        out_type=jax.ShapeDtypeStruct((B, D), jnp.float32),
        mesh=mesh,
        scratch_types=[
            pltpu.VMEM((per_w,), jnp.int32),
            pltpu.VMEM((CH, D), jnp.float32),
            pltpu.SemaphoreType.DMA,
        ],
    )
    def k(table_hbm, idx_hbm, out_hbm, idx_v, rows_v, sem):
        cid = lax.axis_index("c")
        sid = lax.axis_index("s")
        wid = sid * _NC + cid
        base = wid * per_w
        pltpu.sync_copy(idx_hbm.at[pl.ds(base, per_w)], idx_v)
        for j in range(nch):
            pltpu.async_copy(
                table_hbm.at[idx_v.at[pl.ds(j * CH, CH)]], rows_v, sem
            ).wait()
            pltpu.sync_copy(rows_v, out_hbm.at[pl.ds(base + j * CH, CH)])
        if rem:
            pltpu.async_copy(
                table_hbm.at[idx_v.at[pl.ds(nch * CH, rem)]],
                rows_v.at[pl.ds(0, rem)], sem
            ).wait()
            pltpu.sync_copy(rows_v.at[pl.ds(0, rem)],
                            out_hbm.at[pl.ds(base + nch * CH, rem)])

    return k(node_emb, idx_pad)


def _sc_aggregate(h, emb, row, col):
    N, D = h.shape
    E, _ = emb.shape
    EPW = E // _NW
    HPW = EPW // 2
    C = 40
    nch = HPW // C
    assert nch * C == HPW and nch % 2 == 1 and nch >= 5
    gend = (nch - 5) // 2 + 1
    rps = (N // _NS) & ~7
    rem_n = N - rps * _NS
    ND2 = D // 16
    mesh = plsc.VectorSubcoreMesh(core_axis_name="c", subcore_axis_name="s")

    @functools.partial(
        pl.kernel,
        out_type=jax.ShapeDtypeStruct((_NC, N, D), jnp.float32),
        mesh=mesh,
        scratch_types=[
            pltpu.VMEM((HPW,), jnp.int32),
            pltpu.VMEM((HPW,), jnp.int32),
            pltpu.VMEM((C, D), jnp.float32),
            pltpu.VMEM((C, D), jnp.float32),
            pltpu.VMEM((C, D), jnp.float32),
            pltpu.VMEM((C, D), jnp.float32),
            pltpu.VMEM((C, D), jnp.float32),
            pltpu.VMEM((C, D), jnp.float32),
            pltpu.VMEM_SHARED((N, D), jnp.float32),
            pltpu.SemaphoreType.DMA,
            pltpu.SemaphoreType.DMA,
            pltpu.SemaphoreType.DMA,
            pltpu.SemaphoreType.DMA,
        ],
    )
    def k(h_hbm, emb_hbm, row_hbm, col_hbm, out_hbm,
          row_v, col_v, h0, h1, e0, e1, o0, o1, agg_sh,
          semEH0, semEH1, semS0, semS1):
        cid = lax.axis_index("c")
        sid = lax.axis_index("s")
        wid = sid * _NC + cid
        ebase = wid * EPW
        hs = (h0, h1)
        es = (e0, e1)
        os_ = (o0, o1)
        semEH = (semEH0, semEH1)
        semS = (semS0, semS1)

        def load_idx(hoff):
            pltpu.sync_copy(row_hbm.at[pl.ds(ebase + hoff, HPW)], row_v)
            pltpu.sync_copy(col_hbm.at[pl.ds(ebase + hoff, HPW)], col_v)

        def issue(hoff, i, s):
            pltpu.async_copy(emb_hbm.at[pl.ds(ebase + hoff + i * C, C), :],
                             es[s], semEH[s])
            pltpu.async_copy(h_hbm.at[row_v.at[pl.ds(i * C, C)]],
                             hs[s], semEH[s])

        def wait_eh(s):
            pltpu.make_async_copy(emb_hbm.at[pl.ds(ebase, C), :],
                                  es[s], semEH[s]).wait()
            pltpu.make_async_copy(h_hbm.at[pl.ds(0, C)], hs[s],
                                  semEH[s]).wait()

        def wait_s(s):
            pltpu.make_async_copy(h_hbm.at[pl.ds(0, C)], os_[s],
                                  semS[s]).wait()

        def compute(s):
            def ebody(e, _):
                for d in range(ND2):
                    sl = pl.ds(d * 16, 16)
                    os_[s][e, sl] = jnp.maximum(es[s][e, sl] + hs[s][e, sl],
                                                0.0)
                return 0
            lax.fori_loop(0, C, ebody, 0)

        def scatter(i, s):
            pltpu.async_copy(os_[s], agg_sh.at[col_v.at[pl.ds(i * C, C)]],
                             semS[s], add=True)

        def pipeline(hoff):
            for s in range(2):
                wait_eh(s)
                compute(s)
                issue(hoff, 2 + s, s)
                scatter(s, s)

            def pbody(g, _):
                i = 2 * g
                for s in range(2):
                    wait_eh(s)
                    wait_s(s)
                    compute(s)
                    issue(hoff, i + 2 + s, s)
                    scatter(i + s, s)
                return 0
            lax.fori_loop(1, gend, pbody, 0)

            wait_eh(0)
            wait_s(0)
            compute(0)
            issue(hoff, nch - 1, 0)
            scatter(nch - 3, 0)
            wait_eh(1)
            wait_s(1)
            compute(1)
            scatter(nch - 2, 1)
            wait_eh(0)
            wait_s(0)
            compute(0)
            scatter(nch - 1, 0)
            for s in range(2):
                wait_s(s)

        load_idx(0)
        issue(0, 0, 0)
        issue(0, 1, 1)

        def zbody(e, _):
            for d in range(ND2):
                o0[e, pl.ds(d * 16, 16)] = jnp.zeros((16,), jnp.float32)
            return 0
        lax.fori_loop(0, C, zbody, 0)
        nzc = rps // C
        for j in range(nzc):
            pltpu.sync_copy(o0, agg_sh.at[pl.ds(sid * rps + j * C, C), :])
        zrem = rps - nzc * C
        if zrem:
            pltpu.sync_copy(o0.at[pl.ds(0, zrem), :],
                            agg_sh.at[pl.ds(sid * rps + nzc * C, zrem), :])
        if rem_n:
            @pl.when(sid == _NS - 1)
            def _():
                pltpu.sync_copy(o0.at[pl.ds(0, rem_n), :],
                                agg_sh.at[pl.ds(_NS * rps, rem_n), :])
        plsc.subcore_barrier()

        pipeline(0)

        load_idx(HPW)
        issue(HPW, 0, 0)
        issue(HPW, 1, 1)
        pipeline(HPW)
        plsc.subcore_barrier()

        for j in range(nzc):
            pltpu.sync_copy(agg_sh.at[pl.ds(sid * rps + j * C, C), :],
                            out_hbm.at[cid, pl.ds(sid * rps + j * C, C), :])
        if zrem:
            pltpu.sync_copy(
                agg_sh.at[pl.ds(sid * rps + nzc * C, zrem), :],
                out_hbm.at[cid, pl.ds(sid * rps + nzc * C, zrem), :])
        if rem_n:
            @pl.when(sid == _NS - 1)
            def _():
                pltpu.sync_copy(
                    agg_sh.at[pl.ds(_NS * rps, rem_n), :],
                    out_hbm.at[cid, pl.ds(_NS * rps, rem_n), :])

    return k(h, emb, row, col)


def kernel(x, edge_index, edge_attr, batch, node_emb, We, be, eps,
           W1, b1, g1, bt1, m1, v1, W2, b2, gbn, bbn, mbn, vbn):
    N = x.shape[0]
    L, K, D = We.shape
    E = edge_index.shape[1]

    row = edge_index[0].astype(jnp.int32)
    col = edge_index[1].astype(jnp.int32)

    B = ((N + 8 * _NW - 1) // (8 * _NW)) * (8 * _NW)
    xi = jnp.pad(x[:, 0].astype(jnp.int32), (0, B - N))
    h = _sc_embed(node_emb.astype(jnp.float32), xi, B)[:N]

    attr_bf = edge_attr.astype(jnp.bfloat16)
    embs = [_edge_emb_one(attr_bf, We[l], be[l]) for l in range(L)]

    eps2 = eps.reshape(L, 1)
    for l in range(L):
        agg2 = _sc_aggregate(h, embs[l], row, col)
        h = _mlp_layer(h, agg2, eps2, W1, b1, g1, bt1, m1, v1,
                       W2, b2, gbn, bbn, mbn, vbn, l, last=(l == L - 1))
    return h


# speedup vs baseline: 1.3694x; 1.0091x over previous
import functools

import jax
import jax.numpy as jnp
from jax import lax
from jax.experimental import pallas as pl
from jax.experimental.pallas import tpu as pltpu
from jax.experimental.pallas import tpu_sc as plsc

_NC = 2
_NS = 16
_NW = _NC * _NS


def _edge_emb_one(edge_attr, We_l, be_l):
    E, K = edge_attr.shape
    D = We_l.shape[1]
    BE = 8000
    nblk = E // BE

    def body(a_ref, w_ref, b_ref, o_ref):
        a = a_ref[...].astype(jnp.float32)
        o_ref[...] = (
            jnp.dot(a, w_ref[...], preferred_element_type=jnp.float32)
            + b_ref[0]
        )

    return pl.pallas_call(
        body,
        grid=(nblk,),
        in_specs=[
            pl.BlockSpec((BE, K), lambda i: (i, 0)),
            pl.BlockSpec((K, D), lambda i: (0, 0)),
            pl.BlockSpec((1, D), lambda i: (0, 0)),
        ],
        out_specs=pl.BlockSpec((BE, D), lambda i: (i, 0)),
        out_shape=jax.ShapeDtypeStruct((E, D), jnp.float32),
    )(edge_attr, We_l, be_l.reshape(1, D))


def _mlp_layer(h, agg2, eps2, W1, b1, g1, bt1, m1, v1,
               W2, b2, gbn, bbn, mbn, vbn, l, last):
    N, D = h.shape
    H = W1.shape[2]
    BN = 2000
    nblk = N // BN
    eps2_shape = eps2.shape

    def body(h_ref, a_ref, e_ref, w1_ref, b1_ref, g1_ref, bt1_ref, m1_ref,
             v1_ref, w2_ref, b2_ref, gbn_ref, bbn_ref, mbn_ref, vbn_ref,
             o_ref):
        s1 = g1_ref[l] * lax.rsqrt(v1_ref[l] + 1e-5)
        c1 = (b1_ref[l] - m1_ref[l]) * s1 + bt1_ref[l]
        s2 = gbn_ref[l] * lax.rsqrt(vbn_ref[l] + 1e-5)
        c2 = (b2_ref[l] - mbn_ref[l]) * s2 + bbn_ref[l]
        t = h_ref[...] * (1.0 + e_ref[l, 0]) + a_ref[0] + a_ref[1]
        t = jnp.dot(t, w1_ref[0], preferred_element_type=jnp.float32)
        t = jnp.maximum(t * s1 + c1, 0.0)
        t = jnp.dot(t, w2_ref[0], preferred_element_type=jnp.float32)
        t = t * s2 + c2
        if not last:
            t = jnp.maximum(t, 0.0)
        o_ref[...] = t

    L = W1.shape[0]
    full = lambda i: (0, 0)
    return pl.pallas_call(
        body,
        grid=(nblk,),
        in_specs=[
            pl.BlockSpec((BN, D), lambda i: (i, 0)),
            pl.BlockSpec((2, BN, D), lambda i: (0, i, 0)),
            pl.BlockSpec(eps2_shape, full),
            pl.BlockSpec((1, D, H), lambda i: (l, 0, 0)),
            pl.BlockSpec((L, H), full),
            pl.BlockSpec((L, H), full),
            pl.BlockSpec((L, H), full),
            pl.BlockSpec((L, H), full),
            pl.BlockSpec((L, H), full),
            pl.BlockSpec((1, H, D), lambda i: (l, 0, 0)),
            pl.BlockSpec((L, D), full),
            pl.BlockSpec((L, D), full),
            pl.BlockSpec((L, D), full),
            pl.BlockSpec((L, D), full),
            pl.BlockSpec((L, D), full),
        ],
        out_specs=pl.BlockSpec((BN, D), lambda i: (i, 0)),
        out_shape=jax.ShapeDtypeStruct((N, D), jnp.float32),
    )(h, agg2, eps2, W1, b1, g1, bt1, m1, v1, W2, b2, gbn, bbn, mbn, vbn)


def _sc_embed(node_emb, idx_pad, B, N):
    V, D = node_emb.shape
    per_w = B // _NW
    CH = 80
    nch = per_w // CH
    assert nch * CH == per_w and N % CH == 0
    mesh = plsc.VectorSubcoreMesh(core_axis_name="c", subcore_axis_name="s")

    @functools.partial(
        pl.kernel,
        out_type=jax.ShapeDtypeStruct((N, D), jnp.float32),
        mesh=mesh,
        scratch_types=[
            pltpu.VMEM((per_w,), jnp.int32),
            pltpu.VMEM((CH, D), jnp.float32),
            pltpu.SemaphoreType.DMA,
        ],
    )
    def k(table_hbm, idx_hbm, out_hbm, idx_v, rows_v, sem):
        cid = lax.axis_index("c")
        sid = lax.axis_index("s")
        wid = sid * _NC + cid
        base = wid * per_w
        pltpu.sync_copy(idx_hbm.at[pl.ds(base, per_w)], idx_v)
        for j in range(nch):
            @pl.when(base + j * CH < N)
            def _():
                pltpu.async_copy(
                    table_hbm.at[idx_v.at[pl.ds(j * CH, CH)]], rows_v, sem
                ).wait()
                pltpu.sync_copy(rows_v, out_hbm.at[pl.ds(base + j * CH, CH)])

    return k(node_emb, idx_pad)


def _sc_aggregate(h, emb, row, col):
    N, D = h.shape
    E, _ = emb.shape
    EPW = E // _NW
    HPW = EPW // 2
    C = 40
    nch = HPW // C
    assert nch * C == HPW and nch % 2 == 1 and nch >= 5
    gend = (nch - 5) // 2 + 1
    rps = (N // _NS) & ~7
    rem_n = N - rps * _NS
    ND2 = D // 16
    mesh = plsc.VectorSubcoreMesh(core_axis_name="c", subcore_axis_name="s")

    @functools.partial(
        pl.kernel,
        out_type=jax.ShapeDtypeStruct((_NC, N, D), jnp.float32),
        mesh=mesh,
        scratch_types=[
            pltpu.VMEM((HPW,), jnp.int32),
            pltpu.VMEM((HPW,), jnp.int32),
            pltpu.VMEM((C, D), jnp.float32),
            pltpu.VMEM((C, D), jnp.float32),
            pltpu.VMEM((C, D), jnp.float32),
            pltpu.VMEM((C, D), jnp.float32),
            pltpu.VMEM((C, D), jnp.float32),
            pltpu.VMEM((C, D), jnp.float32),
            pltpu.VMEM_SHARED((N, D), jnp.float32),
            pltpu.SemaphoreType.DMA,
            pltpu.SemaphoreType.DMA,
            pltpu.SemaphoreType.DMA,
            pltpu.SemaphoreType.DMA,
        ],
    )
    def k(h_hbm, emb_hbm, row_hbm, col_hbm, out_hbm,
          row_v, col_v, h0, h1, e0, e1, o0, o1, agg_sh,
          semEH0, semEH1, semS0, semS1):
        cid = lax.axis_index("c")
        sid = lax.axis_index("s")
        wid = sid * _NC + cid
        ebase = wid * EPW
        hs = (h0, h1)
        es = (e0, e1)
        os_ = (o0, o1)
        semEH = (semEH0, semEH1)
        semS = (semS0, semS1)

        def load_idx(hoff):
            pltpu.sync_copy(row_hbm.at[pl.ds(ebase + hoff, HPW)], row_v)
            pltpu.sync_copy(col_hbm.at[pl.ds(ebase + hoff, HPW)], col_v)

        def issue(hoff, i, s):
            pltpu.async_copy(emb_hbm.at[pl.ds(ebase + hoff + i * C, C), :],
                             es[s], semEH[s])
            pltpu.async_copy(h_hbm.at[row_v.at[pl.ds(i * C, C)]],
                             hs[s], semEH[s])

        def wait_eh(s):
            pltpu.make_async_copy(emb_hbm.at[pl.ds(ebase, C), :],
                                  es[s], semEH[s]).wait()
            pltpu.make_async_copy(h_hbm.at[pl.ds(0, C)], hs[s],
                                  semEH[s]).wait()

        def wait_s(s):
            pltpu.make_async_copy(h_hbm.at[pl.ds(0, C)], os_[s],
                                  semS[s]).wait()

        def compute(s):
            def ebody(e, _):
                for d in range(ND2):
                    sl = pl.ds(d * 16, 16)
                    os_[s][e, sl] = jnp.maximum(es[s][e, sl] + hs[s][e, sl],
                                                0.0)
                return 0
            lax.fori_loop(0, C, ebody, 0)

        def scatter(i, s):
            pltpu.async_copy(os_[s], agg_sh.at[col_v.at[pl.ds(i * C, C)]],
                             semS[s], add=True)

        def pipeline(hoff):
            for s in range(2):
                wait_eh(s)
                compute(s)
                issue(hoff, 2 + s, s)
                scatter(s, s)

            def pbody(g, _):
                i = 2 * g
                for s in range(2):
                    wait_eh(s)
                    wait_s(s)
                    compute(s)
                    issue(hoff, i + 2 + s, s)
                    scatter(i + s, s)
                return 0
            lax.fori_loop(1, gend, pbody, 0)

            wait_eh(0)
            wait_s(0)
            compute(0)
            issue(hoff, nch - 1, 0)
            scatter(nch - 3, 0)
            wait_eh(1)
            wait_s(1)
            compute(1)
            scatter(nch - 2, 1)
            wait_eh(0)
            wait_s(0)
            compute(0)
            scatter(nch - 1, 0)
            for s in range(2):
                wait_s(s)

        load_idx(0)
        issue(0, 0, 0)
        issue(0, 1, 1)

        def zbody(e, _):
            for d in range(ND2):
                o0[e, pl.ds(d * 16, 16)] = jnp.zeros((16,), jnp.float32)
            return 0
        lax.fori_loop(0, C, zbody, 0)
        nzc = rps // C
        for j in range(nzc):
            pltpu.sync_copy(o0, agg_sh.at[pl.ds(sid * rps + j * C, C), :])
        zrem = rps - nzc * C
        if zrem:
            pltpu.sync_copy(o0.at[pl.ds(0, zrem), :],
                            agg_sh.at[pl.ds(sid * rps + nzc * C, zrem), :])
        if rem_n:
            @pl.when(sid == _NS - 1)
            def _():
                pltpu.sync_copy(o0.at[pl.ds(0, rem_n), :],
                                agg_sh.at[pl.ds(_NS * rps, rem_n), :])
        plsc.subcore_barrier()

        pipeline(0)

        load_idx(HPW)
        issue(HPW, 0, 0)
        issue(HPW, 1, 1)
        pipeline(HPW)
        plsc.subcore_barrier()

        for j in range(nzc):
            pltpu.sync_copy(agg_sh.at[pl.ds(sid * rps + j * C, C), :],
                            out_hbm.at[cid, pl.ds(sid * rps + j * C, C), :])
        if zrem:
            pltpu.sync_copy(
                agg_sh.at[pl.ds(sid * rps + nzc * C, zrem), :],
                out_hbm.at[cid, pl.ds(sid * rps + nzc * C, zrem), :])
        if rem_n:
            @pl.when(sid == _NS - 1)
            def _():
                pltpu.sync_copy(
                    agg_sh.at[pl.ds(_NS * rps, rem_n), :],
                    out_hbm.at[cid, pl.ds(_NS * rps, rem_n), :])

    return k(h, emb, row, col)


def kernel(x, edge_index, edge_attr, batch, node_emb, We, be, eps,
           W1, b1, g1, bt1, m1, v1, W2, b2, gbn, bbn, mbn, vbn):
    N = x.shape[0]
    L, K, D = We.shape
    E = edge_index.shape[1]

    row = edge_index[0].astype(jnp.int32)
    col = edge_index[1].astype(jnp.int32)

    B = ((N + 8 * _NW - 1) // (8 * _NW)) * (8 * _NW)
    xi = jnp.pad(x[:, 0].astype(jnp.int32), (0, B - N))
    h = _sc_embed(node_emb.astype(jnp.float32), xi, B, N)

    attr_bf = edge_attr.astype(jnp.bfloat16)
    embs = [_edge_emb_one(attr_bf, We[l], be[l]) for l in range(L)]

    eps2 = eps.reshape(L, 1)
    for l in range(L):
        agg2 = _sc_aggregate(h, embs[l], row, col)
        h = _mlp_layer(h, agg2, eps2, W1, b1, g1, bt1, m1, v1,
                       W2, b2, gbn, bbn, mbn, vbn, l, last=(l == L - 1))
    return h
